# Initial kernel scaffold; baseline (speedup 1.0000x reference)
import jax
import jax.numpy as jnp
from jax.experimental import pallas as pl


def kernel(x, a_cache, b_cache, base_weight, b_adapter_ids, b_scaling, ranks):
    raise NotImplementedError("write your pallas kernel here")


# speedup vs baseline: 3.6092x; 3.6092x over previous
import jax
import jax.numpy as jnp
from jax.experimental import pallas as pl
from jax.experimental.pallas import tpu as pltpu

_R = 64
_TS = 1024
_OJ = 512


def _lora_body(ids_ref, scale_ref, rank_ref, x_ref, w_ref, a_ref, b_ref,
               o_ref, xa_ref):
    s = pl.program_id(0)
    j = pl.program_id(1)

    @pl.when(j == 0)
    def _():
        xa = jax.lax.dot_general(
            x_ref[...], a_ref[0],
            dimension_numbers=(((1,), (1,)), ((), ())),
            preferred_element_type=jnp.float32)
        col = jax.lax.broadcasted_iota(jnp.int32, (1, _R), 1)
        mask = (col < rank_ref[s]).astype(jnp.float32)
        xa_ref[...] = xa * (mask * scale_ref[s])

    base = jax.lax.dot_general(
        x_ref[...], w_ref[...],
        dimension_numbers=(((1,), (1,)), ((), ())),
        preferred_element_type=jnp.float32)
    lora = jax.lax.dot_general(
        xa_ref[...], b_ref[0],
        dimension_numbers=(((1,), (0,)), ((), ())),
        preferred_element_type=jnp.float32)
    o_ref[...] = base + lora


def kernel(x, a_cache, b_cache, base_weight, b_adapter_ids, b_scaling, ranks):
    T, D = x.shape
    O = base_weight.shape[0]
    n_s = T // _TS
    n_j = O // _OJ
    seq_len = T // b_adapter_ids.shape[0]

    blk_seq = (jnp.arange(n_s, dtype=jnp.int32) * _TS) // seq_len
    ids_blk = b_adapter_ids[blk_seq].astype(jnp.int32)
    scale_blk = b_scaling[blk_seq].astype(jnp.float32)
    rank_blk = ranks[b_adapter_ids][blk_seq].astype(jnp.int32)

    grid_spec = pltpu.PrefetchScalarGridSpec(
        num_scalar_prefetch=3,
        grid=(n_s, n_j),
        in_specs=[
            pl.BlockSpec((_TS, D), lambda s, j, ids, sc, rk: (s, 0)),
            pl.BlockSpec((_OJ, D), lambda s, j, ids, sc, rk: (j, 0)),
            pl.BlockSpec((1, _R, D), lambda s, j, ids, sc, rk: (ids[s], 0, 0)),
            pl.BlockSpec((1, _R, _OJ), lambda s, j, ids, sc, rk: (ids[s], 0, j)),
        ],
        out_specs=pl.BlockSpec((_TS, _OJ), lambda s, j, ids, sc, rk: (s, j)),
        scratch_shapes=[pltpu.VMEM((_TS, _R), jnp.float32)],
    )
    return pl.pallas_call(
        _lora_body,
        grid_spec=grid_spec,
        out_shape=jax.ShapeDtypeStruct((T, O), jnp.float32),
        compiler_params=pltpu.CompilerParams(
            dimension_semantics=("arbitrary", "arbitrary")),
    )(ids_blk, scale_blk, rank_blk, x, base_weight, a_cache, b_cache)
